# baseline (speedup 1.0000x reference)
import jax
import jax.numpy as jnp
from jax import lax
from jax.experimental import pallas as pl
from jax.experimental.pallas import tpu as pltpu

_HID = 128
_NW = 201
_EMB = 16


def _tc_body(pblk, plane, pq, word_ref, spk_ref, w1_ref, b1_ref, w2_ref,
             out_ref, xb_ref, sems):
    cps = []
    for j in range(_NW):
        src = spk_ref if j == 0 else word_ref
        off = pl.multiple_of(pblk[j] * 128, 128)
        cp = pltpu.make_async_copy(
            src.at[:, pl.ds(off, 128)], xb_ref.at[:, pl.ds(j * 128, 128)],
            sems.at[j])
        cp.start()
        cps.append(cp)

    lane = lax.broadcasted_iota(jnp.int32, (1, 128), 1)
    h = jnp.zeros((1, _HID), jnp.float32)
    for j in range(_NW):
        cps[j].wait()
        oh = (lane == plane[j]).astype(jnp.float32)
        blk = xb_ref[:, pl.ds(j * 128, 128)]
        emb = lax.dot_general(blk, oh, (((1,), (1,)), ((), ())),
                              preferred_element_type=jnp.float32)
        seg = w1_ref[pl.ds(_EMB * j, _EMB), :]
        h = h + lax.dot_general(emb, seg, (((0,), (0,)), ((), ())),
                                preferred_element_type=jnp.float32)

    h = h + pq[0] * w1_ref[3216:3217, :] + b1_ref[...]
    h = jnp.maximum(h, 0.0)
    s = jnp.sum(h * w2_ref[...])
    out_ref[...] = jnp.full((1, 1), 1.0 / (1.0 + jnp.exp(-(s + pq[1]))))


def kernel(context_indices, speaker, col_three_indices, quant, sentiment,
           word_emb, speaker_emb, W1, b1, W2, b2):
    del sentiment
    ctx = context_indices.astype(jnp.int32)
    c3 = col_three_indices.astype(jnp.int32)
    widx = jnp.concatenate([speaker.astype(jnp.int32), ctx, c3])
    pblk = widx // 128
    plane = widx % 128
    pq = jnp.concatenate([quant.astype(jnp.float32), b2.astype(jnp.float32)])

    wordT = word_emb.T
    spkT = speaker_emb.T
    w1T = W1.T
    b1r = b1.reshape(1, _HID)

    grid_spec = pltpu.PrefetchScalarGridSpec(
        num_scalar_prefetch=3,
        grid=(1,),
        in_specs=[
            pl.BlockSpec(memory_space=pl.ANY),
            pl.BlockSpec(memory_space=pl.ANY),
            pl.BlockSpec((3217, _HID), lambda i, *_: (0, 0)),
            pl.BlockSpec((1, _HID), lambda i, *_: (0, 0)),
            pl.BlockSpec((1, _HID), lambda i, *_: (0, 0)),
        ],
        out_specs=pl.BlockSpec((1, 1), lambda i, *_: (0, 0)),
        scratch_shapes=[
            pltpu.VMEM((_EMB, _NW * 128), jnp.float32),
            pltpu.SemaphoreType.DMA((_NW,)),
        ],
    )
    out = pl.pallas_call(
        _tc_body,
        grid_spec=grid_spec,
        out_shape=jax.ShapeDtypeStruct((1, 1), jnp.float32),
    )(pblk, plane, pq, wordT, spkT, w1T, b1r, W2)
    return out
